# baseline (speedup 1.0000x reference)
import functools

import jax
import jax.numpy as jnp
from jax import lax
from jax.experimental import pallas as pl
from jax.experimental.pallas import tpu as pltpu
from jax.experimental.pallas import tpu_sc as plsc

N_ATOMS_ = 10000
N_BONDS_ = 320000
ATOM_FDIM_ = 128
BOND_FDIM_ = 144
H = 128
MAX_NB_ = 32
N_MOLS_ = 500

NC, NS = 2, 16
GW = 128

A_PAD = 10240
APC = A_PAD // NC
APS = APC // NS
AWIN = APS * MAX_NB_ // GW

B_PAD = 323584
WPW = (B_PAD // GW) // (NC * NS)

_mesh = plsc.VectorSubcoreMesh(core_axis_name="c", subcore_axis_name="s")


def _gsum_body(msg_hbm, a2b_hbm, sidx_hbm, zeros_hbm, out_hbm,
               gidx_v, sidx_v, gbuf_v, accum_sh):
    c = lax.axis_index("c")
    s = lax.axis_index("s")
    atom_base = c * APC + s * APS
    pltpu.sync_copy(zeros_hbm, accum_sh.at[pl.ds(s * APS, APS)])
    flat_base = atom_base * MAX_NB_

    @pl.loop(0, AWIN)
    def _(w):
        off = flat_base + w * GW
        pltpu.sync_copy(a2b_hbm.at[pl.ds(off, GW)], gidx_v)
        pltpu.sync_copy(sidx_hbm.at[pl.ds(off, GW)], sidx_v)
        pltpu.sync_copy(msg_hbm.at[gidx_v], gbuf_v)
        pltpu.sync_copy(gbuf_v, accum_sh.at[sidx_v], add=True)

    pltpu.sync_copy(accum_sh.at[pl.ds(s * APS, APS)],
                    out_hbm.at[pl.ds(atom_base, APS)])


@jax.jit
def _sc_gather_sum(message, a2b_flat, sidx_flat, zeros_tile):
    k = pl.kernel(
        _gsum_body,
        out_type=jax.ShapeDtypeStruct((A_PAD, H), jnp.float32),
        mesh=_mesh,
        scratch_types=[
            pltpu.VMEM((GW,), jnp.int32),
            pltpu.VMEM((GW,), jnp.int32),
            pltpu.VMEM((GW, H), jnp.float32),
            pltpu.VMEM_SHARED((APC, H), jnp.float32),
        ],
    )
    return k(message, a2b_flat, sidx_flat, zeros_tile)


def _edge_body(msg_hbm, amsg_hbm, brev_hbm, b2a_hbm, out_hbm,
               i1_v, i2_v, b1_v, b2_v):
    c = lax.axis_index("c")
    s = lax.axis_index("s")
    wid = c * NS + s
    base = wid * WPW * GW

    @pl.loop(0, WPW)
    def _(w):
        off = base + w * GW
        pltpu.sync_copy(brev_hbm.at[pl.ds(off, GW)], i1_v)
        pltpu.sync_copy(b2a_hbm.at[pl.ds(off, GW)], i2_v)
        pltpu.sync_copy(msg_hbm.at[i1_v], b1_v)
        pltpu.sync_copy(amsg_hbm.at[i2_v], b2_v)

        @pl.loop(0, GW)
        def _(r):
            for ch in range(H // 16):
                sl = pl.ds(ch * 16, 16)
                b2_v[r, sl] = b2_v[r, sl] - b1_v[r, sl]

        pltpu.sync_copy(b2_v, out_hbm.at[pl.ds(off, GW)])


@jax.jit
def _sc_edge(message, a_msg, brev_p, b2a_p):
    k = pl.kernel(
        _edge_body,
        out_type=jax.ShapeDtypeStruct((B_PAD, H), jnp.float32),
        mesh=_mesh,
        scratch_types=[
            pltpu.VMEM((GW,), jnp.int32),
            pltpu.VMEM((GW,), jnp.int32),
            pltpu.VMEM((GW, H), jnp.float32),
            pltpu.VMEM((GW, H), jnp.float32),
        ],
    )
    return k(message, a_msg, brev_p, b2a_p)


_TB = 512
_NBLK = N_BONDS_ // _TB


def _init_body(fb_ref, wi_ref, inp_ref, msg_ref):
    x = jnp.dot(fb_ref[...], wi_ref[...], preferred_element_type=jnp.float32)
    inp_ref[...] = x
    msg_ref[...] = jnp.maximum(x, 0.0)


@jax.jit
def _tc_init(f_bonds, W_i):
    return pl.pallas_call(
        _init_body,
        grid=(_NBLK,),
        in_specs=[
            pl.BlockSpec((_TB, BOND_FDIM_), lambda i: (i, 0)),
            pl.BlockSpec((BOND_FDIM_, H), lambda i: (0, 0)),
        ],
        out_specs=[
            pl.BlockSpec((_TB, H), lambda i: (i, 0)),
            pl.BlockSpec((_TB, H), lambda i: (i, 0)),
        ],
        out_shape=[
            jax.ShapeDtypeStruct((N_BONDS_, H), jnp.float32),
            jax.ShapeDtypeStruct((N_BONDS_, H), jnp.float32),
        ],
    )(f_bonds, W_i)


def _update_body(inp_ref, pre_ref, wh_ref, out_ref):
    x = jnp.dot(pre_ref[...], wh_ref[...], preferred_element_type=jnp.float32)
    out_ref[...] = jnp.maximum(inp_ref[...] + x, 0.0)


@jax.jit
def _tc_update(inp, pre, W_h):
    return pl.pallas_call(
        _update_body,
        grid=(_NBLK,),
        in_specs=[
            pl.BlockSpec((_TB, H), lambda i: (i, 0)),
            pl.BlockSpec((_TB, H), lambda i: (i, 0)),
            pl.BlockSpec((H, H), lambda i: (0, 0)),
        ],
        out_specs=pl.BlockSpec((_TB, H), lambda i: (i, 0)),
        out_shape=jax.ShapeDtypeStruct((N_BONDS_, H), jnp.float32),
    )(inp, pre, W_h)


_RB = 1024
_RN = A_PAD // _RB
_SEG = 512


def _readout_body(fa_ref, am_ref, mid_ref, woa_ref, woh_ref, bo_ref,
                  out_ref, sums_scr, cnts_scr):
    i = pl.program_id(0)
    hid = (
        jnp.dot(fa_ref[...], woa_ref[...], preferred_element_type=jnp.float32)
        + jnp.dot(am_ref[...], woh_ref[...], preferred_element_type=jnp.float32)
        + bo_ref[...]
    )
    ids = mid_ref[0]
    seg_iota = lax.broadcasted_iota(jnp.int32, (_SEG, _RB), 0)
    onehot_t = (ids == seg_iota).astype(jnp.float32)
    contrib = jnp.dot(onehot_t, hid, preferred_element_type=jnp.float32)
    cnts = jnp.dot(onehot_t, jnp.ones((_RB, H), jnp.float32),
                   preferred_element_type=jnp.float32)

    @pl.when(i == 0)
    def _():
        sums_scr[...] = jnp.zeros_like(sums_scr)
        cnts_scr[...] = jnp.zeros_like(cnts_scr)

    sums_scr[...] += contrib
    cnts_scr[...] += cnts

    @pl.when(i == _RN - 1)
    def _():
        out_ref[...] = sums_scr[...] / jnp.maximum(cnts_scr[...], 1.0)


@jax.jit
def _tc_readout(fa_p, a_msg, mid_r, Wo_a, Wo_h, bo_r):
    return pl.pallas_call(
        _readout_body,
        grid=(_RN,),
        in_specs=[
            pl.BlockSpec((_RB, ATOM_FDIM_), lambda i: (i, 0)),
            pl.BlockSpec((_RB, H), lambda i: (i, 0)),
            pl.BlockSpec((1, 1, _RB), lambda i: (i, 0, 0)),
            pl.BlockSpec((ATOM_FDIM_, H), lambda i: (0, 0)),
            pl.BlockSpec((H, H), lambda i: (0, 0)),
            pl.BlockSpec((1, H), lambda i: (0, 0)),
        ],
        out_specs=pl.BlockSpec((_SEG, H), lambda i: (0, 0)),
        out_shape=jax.ShapeDtypeStruct((_SEG, H), jnp.float32),
        scratch_shapes=[
            pltpu.VMEM((_SEG, H), jnp.float32),
            pltpu.VMEM((_SEG, H), jnp.float32),
        ],
    )(fa_p, a_msg, mid_r, Wo_a, Wo_h, bo_r)


def kernel(f_atoms, f_bonds, a2b, b2a, b2revb, mol_ids, W_i, W_h, W_o, b_o):
    a2b_flat = jnp.pad(a2b, ((0, A_PAD - N_ATOMS_), (0, 0))).reshape(-1)
    sidx_flat = jnp.repeat(
        jnp.arange(A_PAD, dtype=jnp.int32) % APC, MAX_NB_)
    zeros_tile = jnp.zeros((APS, H), jnp.float32)
    b2a_p = jnp.pad(b2a, (0, B_PAD - N_BONDS_))
    brev_p = jnp.pad(b2revb, (0, B_PAD - N_BONDS_))
    fa_p = jnp.pad(f_atoms, ((0, A_PAD - N_ATOMS_), (0, 0)))
    mid_r = jnp.pad(mol_ids, (0, A_PAD - N_ATOMS_),
                    constant_values=N_MOLS_).reshape(_RN, 1, _RB)
    Wo_a = W_o[:ATOM_FDIM_]
    Wo_h = W_o[ATOM_FDIM_:]
    bo_r = b_o.reshape(1, H)

    inp, message = _tc_init(f_bonds, W_i)
    for _ in range(2):
        a_msg = _sc_gather_sum(message, a2b_flat, sidx_flat, zeros_tile)
        pre = _sc_edge(message, a_msg, brev_p, b2a_p)
        message = _tc_update(inp, pre, W_h)
    a_msg = _sc_gather_sum(message, a2b_flat, sidx_flat, zeros_tile)
    out = _tc_readout(fa_p, a_msg, mid_r, Wo_a, Wo_h, bo_r)
    return out[:N_MOLS_]


# speedup vs baseline: 1.3116x; 1.3116x over previous
import functools

import jax
import jax.numpy as jnp
from jax import lax
from jax.experimental import pallas as pl
from jax.experimental.pallas import tpu as pltpu
from jax.experimental.pallas import tpu_sc as plsc

N_ATOMS_ = 10000
N_BONDS_ = 320000
ATOM_FDIM_ = 128
BOND_FDIM_ = 144
H = 128
MAX_NB_ = 32
N_MOLS_ = 500

NC, NS = 2, 16
GW = 128

A_PAD = 10240
APC = A_PAD // NC
APS = APC // NS
AWIN = APS * MAX_NB_ // GW

B_PAD = 323584
WPW = (B_PAD // GW) // (NC * NS)

_mesh = plsc.VectorSubcoreMesh(core_axis_name="c", subcore_axis_name="s")


_ANB = 3
_RCH = 64
_NR = APS // _RCH
_RWIN = _RCH * MAX_NB_ // GW


def _gsum_body(msg_hbm, a2b_hbm, sidx_hbm, zeros_hbm, out_hbm,
               gidx_v, sidx_v, gbufs_v, accum_sh, *sems):
    gsem = sems[:_ANB]
    ssem = sems[_ANB:]
    c = lax.axis_index("c")
    s = lax.axis_index("s")
    atom_base = pl.multiple_of(c * APC + s * APS, APS)
    row_base = pl.multiple_of(atom_base // 4, AWIN)
    stripe = pl.multiple_of(s * _RCH, _RCH)
    pltpu.sync_copy(a2b_hbm.at[pl.ds(row_base, AWIN)], gidx_v)
    pltpu.sync_copy(sidx_hbm.at[pl.ds(row_base, AWIN)], sidx_v)

    def start_g(w, k):
        pltpu.async_copy(msg_hbm.at[gidx_v.at[w, 0]], gbufs_v.at[k], gsem[k])

    def wait_g(w, k):
        pltpu.make_async_copy(msg_hbm.at[gidx_v.at[w, 0]], gbufs_v.at[k],
                              gsem[k]).wait()

    def do_scat(w, k):
        pltpu.async_copy(gbufs_v.at[k], accum_sh.at[sidx_v.at[w, 0]], ssem[k],
                         add=True)
        pltpu.make_async_copy(gbufs_v.at[k], accum_sh.at[sidx_v.at[w, 0]],
                              ssem[k]).wait()

    @pl.loop(0, _NR)
    def _(r):
        w0 = r * _RWIN
        pltpu.sync_copy(zeros_hbm, accum_sh.at[pl.ds(stripe, _RCH)])
        start_g(w0 + 0, 0)
        start_g(w0 + 1, 1)
        for t in range(_RWIN):
            j = t % _ANB
            wait_g(w0 + t, j)
            do_scat(w0 + t, j)
            if t + 2 < _RWIN:
                start_g(w0 + t + 2, (t + 2) % _ANB)
        pltpu.sync_copy(accum_sh.at[pl.ds(stripe, _RCH)],
                        out_hbm.at[pl.ds(atom_base + r * _RCH, _RCH)])


@jax.jit
def _sc_gather_sum(message, a2b_rows, sidx_rows, zeros_tile):
    k = pl.kernel(
        _gsum_body,
        out_type=jax.ShapeDtypeStruct((A_PAD, H), jnp.float32),
        mesh=_mesh,
        scratch_types=[
            pltpu.VMEM((AWIN, 1, GW), jnp.int32),
            pltpu.VMEM((AWIN, 1, GW), jnp.int32),
            pltpu.VMEM((_ANB, GW, H), jnp.float32),
            pltpu.VMEM_SHARED((NS * _RCH, H), jnp.float32),
        ] + [pltpu.SemaphoreType.DMA] * (2 * _ANB),
    )
    return k(message, a2b_rows, sidx_rows, zeros_tile)


_BNB = 3


def _edge_body(msg_hbm, amsg_hbm, brev_hbm, b2a_hbm, out_hbm,
               i1_v, i2_v, b1s_v, b2s_v, *sems):
    g1sem = sems[:_BNB]
    g2sem = sems[_BNB:2 * _BNB]
    stsem = sems[2 * _BNB:]
    c = lax.axis_index("c")
    s = lax.axis_index("s")
    wid = c * NS + s
    base = pl.multiple_of(wid * WPW * GW, WPW * GW)
    row_base = wid * WPW
    pltpu.sync_copy(brev_hbm.at[pl.ds(row_base, WPW)], i1_v)
    pltpu.sync_copy(b2a_hbm.at[pl.ds(row_base, WPW)], i2_v)

    def start_g(w, k):
        pltpu.async_copy(msg_hbm.at[i1_v.at[w, 0]], b1s_v.at[k], g1sem[k])
        pltpu.async_copy(amsg_hbm.at[i2_v.at[w, 0]], b2s_v.at[k], g2sem[k])

    def wait_g(w, k):
        pltpu.make_async_copy(msg_hbm.at[i1_v.at[w, 0]], b1s_v.at[k],
                              g1sem[k]).wait()
        pltpu.make_async_copy(amsg_hbm.at[i2_v.at[w, 0]], b2s_v.at[k],
                              g2sem[k]).wait()

    def start_st(w, k):
        pltpu.async_copy(b2s_v.at[k], out_hbm.at[pl.ds(pl.multiple_of(base + w * GW, GW), GW)],
                         stsem[k])

    def wait_st(w, k):
        pltpu.make_async_copy(b2s_v.at[k],
                              out_hbm.at[pl.ds(pl.multiple_of(base + w * GW, GW), GW)],
                              stsem[k]).wait()

    def sub(k):
        @pl.loop(0, GW)
        def _(r):
            for ch in range(H // 16):
                sl = pl.ds(ch * 16, 16)
                b2s_v[k, r, sl] = b2s_v[k, r, sl] - b1s_v[k, r, sl]

    start_g(0, 0)
    start_g(1, 1)
    start_g(2, 2)
    wait_g(0, 0)
    sub(0)
    start_st(0, 0)
    wait_st(0, 0)
    start_g(3, 0)
    wait_g(1, 1)
    sub(1)
    start_st(1, 1)

    @pl.loop(0, 25)
    def _(i):
        for k in range(3):
            w = 2 + i * 3 + k
            j = (2 + k) % 3
            j2 = (1 + k) % 3
            wait_st(w - 1, j2)
            start_g(w + 2, j2)
            wait_g(w, j)
            sub(j)
            start_st(w, j)

    wait_st(76, 1)
    wait_g(77, 2)
    sub(2)
    start_st(77, 2)
    wait_g(78, 0)
    sub(0)
    start_st(78, 0)
    wait_st(77, 2)
    wait_st(78, 0)


@jax.jit
def _sc_edge(message, a_msg, brev_rows, b2a_rows):
    k = pl.kernel(
        _edge_body,
        out_type=jax.ShapeDtypeStruct((B_PAD, H), jnp.float32),
        mesh=_mesh,
        scratch_types=[
            pltpu.VMEM((WPW, 1, GW), jnp.int32),
            pltpu.VMEM((WPW, 1, GW), jnp.int32),
            pltpu.VMEM((_BNB, GW, H), jnp.float32),
            pltpu.VMEM((_BNB, GW, H), jnp.float32),
        ] + [pltpu.SemaphoreType.DMA] * (3 * _BNB),
    )
    return k(message, a_msg, brev_rows, b2a_rows)


_TB = 512
_NBLK = N_BONDS_ // _TB


def _init_body(fb_ref, wi_ref, inp_ref, msg_ref):
    x = jnp.dot(fb_ref[...], wi_ref[...], preferred_element_type=jnp.float32)
    inp_ref[...] = x
    msg_ref[...] = jnp.maximum(x, 0.0)


@jax.jit
def _tc_init(f_bonds, W_i):
    return pl.pallas_call(
        _init_body,
        grid=(_NBLK,),
        in_specs=[
            pl.BlockSpec((_TB, BOND_FDIM_), lambda i: (i, 0)),
            pl.BlockSpec((BOND_FDIM_, H), lambda i: (0, 0)),
        ],
        out_specs=[
            pl.BlockSpec((_TB, H), lambda i: (i, 0)),
            pl.BlockSpec((_TB, H), lambda i: (i, 0)),
        ],
        out_shape=[
            jax.ShapeDtypeStruct((N_BONDS_, H), jnp.float32),
            jax.ShapeDtypeStruct((N_BONDS_, H), jnp.float32),
        ],
    )(f_bonds, W_i)


def _update_body(inp_ref, pre_ref, wh_ref, out_ref):
    x = jnp.dot(pre_ref[...], wh_ref[...], preferred_element_type=jnp.float32)
    out_ref[...] = jnp.maximum(inp_ref[...] + x, 0.0)


@jax.jit
def _tc_update(inp, pre, W_h):
    return pl.pallas_call(
        _update_body,
        grid=(_NBLK,),
        in_specs=[
            pl.BlockSpec((_TB, H), lambda i: (i, 0)),
            pl.BlockSpec((_TB, H), lambda i: (i, 0)),
            pl.BlockSpec((H, H), lambda i: (0, 0)),
        ],
        out_specs=pl.BlockSpec((_TB, H), lambda i: (i, 0)),
        out_shape=jax.ShapeDtypeStruct((N_BONDS_, H), jnp.float32),
    )(inp, pre, W_h)


_RB = 1024
_RN = A_PAD // _RB
_SEG = 512


def _readout_body(fa_ref, am_ref, mid_ref, woa_ref, woh_ref, bo_ref,
                  out_ref, sums_scr, cnts_scr):
    i = pl.program_id(0)
    hid = (
        jnp.dot(fa_ref[...], woa_ref[...], preferred_element_type=jnp.float32)
        + jnp.dot(am_ref[...], woh_ref[...], preferred_element_type=jnp.float32)
        + bo_ref[...]
    )
    ids = mid_ref[0]
    seg_iota = lax.broadcasted_iota(jnp.int32, (_SEG, _RB), 0)
    onehot_t = (ids == seg_iota).astype(jnp.float32)
    contrib = jnp.dot(onehot_t, hid, preferred_element_type=jnp.float32)
    cnts = jnp.dot(onehot_t, jnp.ones((_RB, H), jnp.float32),
                   preferred_element_type=jnp.float32)

    @pl.when(i == 0)
    def _():
        sums_scr[...] = jnp.zeros_like(sums_scr)
        cnts_scr[...] = jnp.zeros_like(cnts_scr)

    sums_scr[...] += contrib
    cnts_scr[...] += cnts

    @pl.when(i == _RN - 1)
    def _():
        out_ref[...] = sums_scr[...] / jnp.maximum(cnts_scr[...], 1.0)


@jax.jit
def _tc_readout(fa_p, a_msg, mid_r, Wo_a, Wo_h, bo_r):
    return pl.pallas_call(
        _readout_body,
        grid=(_RN,),
        in_specs=[
            pl.BlockSpec((_RB, ATOM_FDIM_), lambda i: (i, 0)),
            pl.BlockSpec((_RB, H), lambda i: (i, 0)),
            pl.BlockSpec((1, 1, _RB), lambda i: (i, 0, 0)),
            pl.BlockSpec((ATOM_FDIM_, H), lambda i: (0, 0)),
            pl.BlockSpec((H, H), lambda i: (0, 0)),
            pl.BlockSpec((1, H), lambda i: (0, 0)),
        ],
        out_specs=pl.BlockSpec((_SEG, H), lambda i: (0, 0)),
        out_shape=jax.ShapeDtypeStruct((_SEG, H), jnp.float32),
        scratch_shapes=[
            pltpu.VMEM((_SEG, H), jnp.float32),
            pltpu.VMEM((_SEG, H), jnp.float32),
        ],
    )(fa_p, a_msg, mid_r, Wo_a, Wo_h, bo_r)


def kernel(f_atoms, f_bonds, a2b, b2a, b2revb, mol_ids, W_i, W_h, W_o, b_o):
    a2b_rows = jnp.pad(a2b, ((0, A_PAD - N_ATOMS_), (0, 0))).reshape(-1, 1, GW)
    _al = jnp.arange(A_PAD, dtype=jnp.int32)
    sidx_rows = jnp.repeat(
        ((_al % APC) // APS) * _RCH + (_al % _RCH),
        MAX_NB_).reshape(-1, 1, GW)
    zeros_tile = jnp.zeros((_RCH, H), jnp.float32)
    b2a_p = jnp.pad(b2a, (0, B_PAD - N_BONDS_)).reshape(-1, 1, GW)
    brev_p = jnp.pad(b2revb, (0, B_PAD - N_BONDS_)).reshape(-1, 1, GW)
    fa_p = jnp.pad(f_atoms, ((0, A_PAD - N_ATOMS_), (0, 0)))
    mid_r = jnp.pad(mol_ids, (0, A_PAD - N_ATOMS_),
                    constant_values=N_MOLS_).reshape(_RN, 1, _RB)
    Wo_a = W_o[:ATOM_FDIM_]
    Wo_h = W_o[ATOM_FDIM_:]
    bo_r = b_o.reshape(1, H)

    inp, message = _tc_init(f_bonds, W_i)
    for _ in range(2):
        a_msg = _sc_gather_sum(message, a2b_rows, sidx_rows, zeros_tile)
        pre = _sc_edge(message, a_msg, brev_p, b2a_p)
        message = _tc_update(inp, pre, W_h)
    a_msg = _sc_gather_sum(message, a2b_rows, sidx_rows, zeros_tile)
    out = _tc_readout(fa_p, a_msg, mid_r, Wo_a, Wo_h, bo_r)
    return out[:N_MOLS_]


# speedup vs baseline: 1.6554x; 1.2621x over previous
import functools

import jax
import jax.numpy as jnp
from jax import lax
from jax.experimental import pallas as pl
from jax.experimental.pallas import tpu as pltpu
from jax.experimental.pallas import tpu_sc as plsc

N_ATOMS_ = 10000
N_BONDS_ = 320000
ATOM_FDIM_ = 128
BOND_FDIM_ = 144
H = 128
MAX_NB_ = 32
N_MOLS_ = 500

NC, NS = 2, 16
GW = 128

A_PAD = 10240
APC = A_PAD // NC
APS = APC // NS
AWIN = APS * MAX_NB_ // GW

B_PAD = 323584
WPW = (B_PAD // GW) // (NC * NS)

_mesh = plsc.VectorSubcoreMesh(core_axis_name="c", subcore_axis_name="s")


_ANB = 3
_RCH = 64
_NR = APS // _RCH
_RWIN = _RCH * MAX_NB_ // GW


def _gsum_body(msg_hbm, a2b_hbm, sidx_hbm, zeros_hbm, out_hbm,
               gidx_v, sidx_v, gbufs_v, accum_sh, *sems):
    gsem = sems[:_ANB]
    ssem = sems[_ANB:2 * _ANB]
    csem = sems[2 * _ANB]
    c = lax.axis_index("c")
    s = lax.axis_index("s")
    atom_base = pl.multiple_of(c * APC + s * APS, APS)
    flat_base = pl.multiple_of(atom_base * MAX_NB_, APS * MAX_NB_)
    row_base = pl.multiple_of(atom_base // 4, AWIN)
    stripe = pl.multiple_of(s * _RCH, _RCH)
    pltpu.sync_copy(a2b_hbm.at[pl.ds(flat_base, AWIN * GW)], gidx_v)
    pltpu.sync_copy(sidx_hbm.at[pl.ds(row_base, AWIN)], sidx_v)

    def gidx(w):
        return gidx_v.at[pl.ds(w * GW, GW)]

    def start_g(w, k):
        pltpu.async_copy(msg_hbm.at[gidx(w)], gbufs_v.at[k], gsem[k])

    def wait_g(w, k):
        pltpu.make_async_copy(msg_hbm.at[gidx(w)], gbufs_v.at[k],
                              gsem[k]).wait()

    def start_s(w, k):
        pltpu.async_copy(gbufs_v.at[k], accum_sh.at[sidx_v.at[w, 0]], ssem[k],
                         add=True)

    def wait_s(w, k):
        pltpu.make_async_copy(gbufs_v.at[k], accum_sh.at[sidx_v.at[w, 0]],
                              ssem[k]).wait()

    def copyout(r):
        return pltpu.make_async_copy(
            accum_sh.at[pl.ds(stripe, _RCH)],
            out_hbm.at[pl.ds(atom_base + r * _RCH, _RCH)], csem)

    @pl.loop(0, _NR)
    def _(r):
        w0 = r * _RWIN
        for t in range(_ANB):
            start_g(w0 + t, t)
        @pl.when(r > 0)
        def _():
            copyout(r - 1).wait()
        pltpu.sync_copy(zeros_hbm, accum_sh.at[pl.ds(stripe, _RCH)])

        @pl.loop(0, 4)
        def _(i):
            for u in range(3):
                t = i * 3 + u
                wait_g(w0 + t, u)
                start_s(w0 + t, u)

                @pl.when(i > 0)
                def _():
                    wait_s(w0 + t - 3, u)
                start_g(w0 + t + 3, u)

        wait_g(w0 + 12, 0)
        start_s(w0 + 12, 0)
        wait_s(w0 + 9, 0)
        start_g(w0 + 15, 0)
        wait_g(w0 + 13, 1)
        start_s(w0 + 13, 1)
        wait_s(w0 + 10, 1)
        wait_g(w0 + 14, 2)
        start_s(w0 + 14, 2)
        wait_s(w0 + 11, 2)
        wait_g(w0 + 15, 0)
        start_s(w0 + 15, 0)
        wait_s(w0 + 12, 0)
        wait_s(w0 + 13, 1)
        wait_s(w0 + 14, 2)
        wait_s(w0 + 15, 0)
        copyout(r).start()

    copyout(_NR - 1).wait()


@jax.jit
def _sc_gather_sum(message, a2b_flat, sidx_rows, zeros_tile):
    k = pl.kernel(
        _gsum_body,
        out_type=jax.ShapeDtypeStruct((A_PAD, H), jnp.float32),
        mesh=_mesh,
        scratch_types=[
            pltpu.VMEM((AWIN * GW,), jnp.int32),
            pltpu.VMEM((AWIN, 1, GW), jnp.int32),
            pltpu.VMEM((_ANB, GW, H), jnp.float32),
            pltpu.VMEM_SHARED((NS * _RCH, H), jnp.float32),
        ] + [pltpu.SemaphoreType.DMA] * (2 * _ANB + 1),
    )
    return k(message, a2b_flat, sidx_rows, zeros_tile)


_BNB = 3


def _edge_body(msg_hbm, amsg_hbm, brev_hbm, b2a_hbm, out_hbm,
               i1_v, i2_v, b1s_v, b2s_v, *sems):
    g1sem = sems[:_BNB]
    g2sem = sems[_BNB:2 * _BNB]
    stsem = sems[2 * _BNB:]
    c = lax.axis_index("c")
    s = lax.axis_index("s")
    wid = c * NS + s
    base = pl.multiple_of(wid * WPW * GW, WPW * GW)
    pltpu.sync_copy(brev_hbm.at[pl.ds(base, WPW * GW)], i1_v)
    pltpu.sync_copy(b2a_hbm.at[pl.ds(base, WPW * GW)], i2_v)

    def start_g(w, k):
        pltpu.async_copy(msg_hbm.at[i1_v.at[pl.ds(w * GW, GW)]], b1s_v.at[k], g1sem[k])
        pltpu.async_copy(amsg_hbm.at[i2_v.at[pl.ds(w * GW, GW)]], b2s_v.at[k], g2sem[k])

    def wait_g(w, k):
        pltpu.make_async_copy(msg_hbm.at[i1_v.at[pl.ds(w * GW, GW)]], b1s_v.at[k],
                              g1sem[k]).wait()
        pltpu.make_async_copy(amsg_hbm.at[i2_v.at[pl.ds(w * GW, GW)]], b2s_v.at[k],
                              g2sem[k]).wait()

    def start_st(w, k):
        pltpu.async_copy(b2s_v.at[k], out_hbm.at[pl.ds(pl.multiple_of(base + w * GW, GW), GW)],
                         stsem[k])

    def wait_st(w, k):
        pltpu.make_async_copy(b2s_v.at[k],
                              out_hbm.at[pl.ds(pl.multiple_of(base + w * GW, GW), GW)],
                              stsem[k]).wait()

    def sub(k):
        @pl.loop(0, GW, step=4)
        def _(r):
            for dr in range(4):
                for ch in range(H // 16):
                    sl = pl.ds(ch * 16, 16)
                    b2s_v[k, r + dr, sl] = b2s_v[k, r + dr, sl] - b1s_v[k, r + dr, sl]

    start_g(0, 0)
    start_g(1, 1)
    start_g(2, 2)
    wait_g(0, 0)
    sub(0)
    start_st(0, 0)
    wait_st(0, 0)
    start_g(3, 0)
    wait_g(1, 1)
    sub(1)
    start_st(1, 1)

    @pl.loop(0, 25)
    def _(i):
        for k in range(3):
            w = 2 + i * 3 + k
            j = (2 + k) % 3
            j2 = (1 + k) % 3
            wait_st(w - 1, j2)
            start_g(w + 2, j2)
            wait_g(w, j)
            sub(j)
            start_st(w, j)

    wait_st(76, 1)
    wait_g(77, 2)
    sub(2)
    start_st(77, 2)
    wait_g(78, 0)
    sub(0)
    start_st(78, 0)
    wait_st(77, 2)
    wait_st(78, 0)


@jax.jit
def _sc_edge(message, a_msg, brev_rows, b2a_rows):
    k = pl.kernel(
        _edge_body,
        out_type=jax.ShapeDtypeStruct((B_PAD, H), jnp.float32),
        mesh=_mesh,
        scratch_types=[
            pltpu.VMEM((WPW * GW,), jnp.int32),
            pltpu.VMEM((WPW * GW,), jnp.int32),
            pltpu.VMEM((_BNB, GW, H), jnp.float32),
            pltpu.VMEM((_BNB, GW, H), jnp.float32),
        ] + [pltpu.SemaphoreType.DMA] * (3 * _BNB),
    )
    return k(message, a_msg, brev_rows, b2a_rows)


_TB = 3200
_NBLK = N_BONDS_ // _TB


def _init_body(fb_ref, wi_ref, inp_ref, msg_ref):
    x = jnp.dot(fb_ref[...], wi_ref[...], preferred_element_type=jnp.float32)
    inp_ref[...] = x
    msg_ref[...] = jnp.maximum(x, 0.0)


@jax.jit
def _tc_init(f_bonds, W_i):
    return pl.pallas_call(
        _init_body,
        grid=(_NBLK,),
        in_specs=[
            pl.BlockSpec((_TB, BOND_FDIM_), lambda i: (i, 0)),
            pl.BlockSpec((BOND_FDIM_, H), lambda i: (0, 0)),
        ],
        out_specs=[
            pl.BlockSpec((_TB, H), lambda i: (i, 0)),
            pl.BlockSpec((_TB, H), lambda i: (i, 0)),
        ],
        out_shape=[
            jax.ShapeDtypeStruct((N_BONDS_, H), jnp.float32),
            jax.ShapeDtypeStruct((N_BONDS_, H), jnp.float32),
        ],
    )(f_bonds, W_i)


def _update_body(inp_ref, pre_ref, wh_ref, out_ref):
    x = jnp.dot(pre_ref[...], wh_ref[...], preferred_element_type=jnp.float32)
    out_ref[...] = jnp.maximum(inp_ref[...] + x, 0.0)


@jax.jit
def _tc_update(inp, pre, W_h):
    return pl.pallas_call(
        _update_body,
        grid=(_NBLK,),
        in_specs=[
            pl.BlockSpec((_TB, H), lambda i: (i, 0)),
            pl.BlockSpec((_TB, H), lambda i: (i, 0)),
            pl.BlockSpec((H, H), lambda i: (0, 0)),
        ],
        out_specs=pl.BlockSpec((_TB, H), lambda i: (i, 0)),
        out_shape=jax.ShapeDtypeStruct((N_BONDS_, H), jnp.float32),
    )(inp, pre, W_h)


_RB = 1024
_RN = A_PAD // _RB
_SEG = 512


def _readout_body(fa_ref, am_ref, mid_ref, woa_ref, woh_ref, bo_ref,
                  out_ref, sums_scr, cnts_scr):
    i = pl.program_id(0)
    hid = (
        jnp.dot(fa_ref[...], woa_ref[...], preferred_element_type=jnp.float32)
        + jnp.dot(am_ref[...], woh_ref[...], preferred_element_type=jnp.float32)
        + bo_ref[...]
    )
    ids = mid_ref[0]
    seg_iota = lax.broadcasted_iota(jnp.int32, (_SEG, _RB), 0)
    onehot_t = (ids == seg_iota).astype(jnp.float32)
    contrib = jnp.dot(onehot_t, hid, preferred_element_type=jnp.float32)
    cnts = jnp.dot(onehot_t, jnp.ones((_RB, H), jnp.float32),
                   preferred_element_type=jnp.float32)

    @pl.when(i == 0)
    def _():
        sums_scr[...] = jnp.zeros_like(sums_scr)
        cnts_scr[...] = jnp.zeros_like(cnts_scr)

    sums_scr[...] += contrib
    cnts_scr[...] += cnts

    @pl.when(i == _RN - 1)
    def _():
        out_ref[...] = sums_scr[...] / jnp.maximum(cnts_scr[...], 1.0)


@jax.jit
def _tc_readout(fa_p, a_msg, mid_r, Wo_a, Wo_h, bo_r):
    return pl.pallas_call(
        _readout_body,
        grid=(_RN,),
        in_specs=[
            pl.BlockSpec((_RB, ATOM_FDIM_), lambda i: (i, 0)),
            pl.BlockSpec((_RB, H), lambda i: (i, 0)),
            pl.BlockSpec((1, 1, _RB), lambda i: (i, 0, 0)),
            pl.BlockSpec((ATOM_FDIM_, H), lambda i: (0, 0)),
            pl.BlockSpec((H, H), lambda i: (0, 0)),
            pl.BlockSpec((1, H), lambda i: (0, 0)),
        ],
        out_specs=pl.BlockSpec((_SEG, H), lambda i: (0, 0)),
        out_shape=jax.ShapeDtypeStruct((_SEG, H), jnp.float32),
        scratch_shapes=[
            pltpu.VMEM((_SEG, H), jnp.float32),
            pltpu.VMEM((_SEG, H), jnp.float32),
        ],
    )(fa_p, a_msg, mid_r, Wo_a, Wo_h, bo_r)


def kernel(f_atoms, f_bonds, a2b, b2a, b2revb, mol_ids, W_i, W_h, W_o, b_o):
    a2b_flat = jnp.pad(a2b, ((0, A_PAD - N_ATOMS_), (0, 0))).reshape(-1)
    _al = jnp.arange(A_PAD, dtype=jnp.int32)
    sidx_rows = jnp.repeat(
        ((_al % APC) // APS) * _RCH + (_al % _RCH),
        MAX_NB_).reshape(-1, 1, GW)
    zeros_tile = jnp.zeros((_RCH, H), jnp.float32)
    b2a_p = jnp.pad(b2a, (0, B_PAD - N_BONDS_))
    brev_p = jnp.pad(b2revb, (0, B_PAD - N_BONDS_))
    fa_p = jnp.pad(f_atoms, ((0, A_PAD - N_ATOMS_), (0, 0)))
    mid_r = jnp.pad(mol_ids, (0, A_PAD - N_ATOMS_),
                    constant_values=N_MOLS_).reshape(_RN, 1, _RB)
    Wo_a = W_o[:ATOM_FDIM_]
    Wo_h = W_o[ATOM_FDIM_:]
    bo_r = b_o.reshape(1, H)

    inp, message = _tc_init(f_bonds, W_i)
    for _ in range(2):
        a_msg = _sc_gather_sum(message, a2b_flat, sidx_rows, zeros_tile)
        pre = _sc_edge(message, a_msg, brev_p, b2a_p)
        message = _tc_update(inp, pre, W_h)
    a_msg = _sc_gather_sum(message, a2b_flat, sidx_rows, zeros_tile)
    out = _tc_readout(fa_p, a_msg, mid_r, Wo_a, Wo_h, bo_r)
    return out[:N_MOLS_]
